# Initial kernel scaffold; baseline (speedup 1.0000x reference)
import jax
import jax.numpy as jnp
from jax.experimental import pallas as pl


def kernel(x, edge_index, Wl1, bl1, Wr1, Wl2, bl2, Wr2, Wl3, bl3, Wr3):
    raise NotImplementedError("write your pallas kernel here")


# speedup vs baseline: 3.4535x; 3.4535x over previous
import functools

import jax
import jax.numpy as jnp
from jax import lax
from jax.experimental import pallas as pl
from jax.experimental.pallas import tpu as pltpu
from jax.experimental.pallas import tpu_sc as plsc

N = 10000
E = 320000
D = 128
H = 64

NC = 2
NS = 16
NW = NC * NS
NP = 313
N_PAD = NW * NP

CH = 10000
G = 256
CAP = E + 2 * CH

_SRC_MASK = (1 << 14) - 1


def _worker_id():
    return lax.axis_index("s") * NC + lax.axis_index("c")


def _memset_i32(ref, n, value):
    v = jnp.full((16,), value, jnp.int32)

    def body(i, carry):
        ref[pl.ds(i * 16, 16)] = v
        return carry

    lax.fori_loop(0, n // 16, body, 0)


def _bucket_body(dst_hbm, src_hbm, bucket_hbm, counts_hbm,
                 dst_v, src_v, sel_v, cnt_v, sem):
    w = _worker_id()
    lo = w * NP
    hi = lo + NP
    lo_v = jnp.full((16,), lo, jnp.int32)
    hi_v = jnp.full((16,), hi, jnp.int32)
    trash = (NP << 14) | lo
    _memset_i32(sel_v, CH + 16, trash)

    def chunk_body(c, off):
        pltpu.sync_copy(dst_hbm.at[pl.ds(pl.multiple_of(c * CH, 8), CH)], dst_v)
        pltpu.sync_copy(src_hbm.at[pl.ds(pl.multiple_of(c * CH, 8), CH)], src_v)

        def filt(i, cnt):
            d = dst_v[pl.ds(i * 16, 16)]
            s = src_v[pl.ds(i * 16, 16)]
            m = (d >= lo_v) & (d < hi_v)
            packed = ((d - lo_v) << 14) | s
            mi = m.astype(jnp.int32)
            pos = plsc.cumsum(mi) + (cnt - 1)
            plsc.store_scatter(sel_v, [pos], packed, mask=m)
            return cnt + jnp.sum(mi)

        cnt = lax.fori_loop(0, CH // 16, filt, 0)
        sel_v[pl.ds(cnt, 16)] = jnp.full((16,), trash, jnp.int32)
        cnt_p = (cnt + 7) & ~7
        pltpu.sync_copy(sel_v.at[pl.ds(0, CH)],
                        bucket_hbm.at[pl.ds(pl.multiple_of(w * CAP + off, 8), CH)])
        return off + cnt_p

    off = lax.fori_loop(0, E // CH, chunk_body, 0)
    pltpu.sync_copy(sel_v.at[pl.ds(0, CH)], bucket_hbm.at[pl.ds(pl.multiple_of(w * CAP + off, 8), CH)])
    cnt_v[...] = jnp.full((16,), off, jnp.int32)
    pltpu.sync_copy(cnt_v, counts_hbm.at[pl.ds(pl.multiple_of(w * 16, 8), 16)])


@functools.partial(jax.jit, static_argnums=())
def _bucket(dst, src):
    mesh = plsc.VectorSubcoreMesh(core_axis_name="c", subcore_axis_name="s")
    kern = pl.kernel(
        _bucket_body,
        out_type=[
            jax.ShapeDtypeStruct((NW * CAP,), jnp.int32),
            jax.ShapeDtypeStruct((NW * 16,), jnp.int32),
        ],
        mesh=mesh,
        compiler_params=pltpu.CompilerParams(needs_layout_passes=False),
        scratch_types=[
            pltpu.VMEM((CH,), jnp.int32),
            pltpu.VMEM((CH,), jnp.int32),
            pltpu.VMEM((CH + 16,), jnp.int32),
            pltpu.VMEM((16,), jnp.int32),
            pltpu.SemaphoreType.DMA,
        ],
    )
    return kern(dst, src)


def _agg_body(F, h_hbm, bucket_hbm, counts_hbm, out_hbm,
              pk_v, idx_v, loc_v, rows_v, acc_v, cnt_v, sem):
    w = _worker_id()
    pltpu.sync_copy(counts_hbm.at[pl.ds(pl.multiple_of(w * 16, 8), 16)], cnt_v)
    cnt = jnp.max(cnt_v[...])

    neginf = jnp.full((16,), -jnp.inf, jnp.float32)

    def init(i, carry):
        acc_v[pl.ds(i * 16, 16)] = neginf
        return carry

    lax.fori_loop(0, (NP + 1) * F // 16, init, 0)

    nb = lax.div(cnt + (G - 1), G)

    def batch(b, carry):
        pltpu.sync_copy(bucket_hbm.at[pl.ds(pl.multiple_of(w * CAP + b * G, 8), G)], pk_v)
        for j in range(G // 16):
            p = pk_v[pl.ds(j * 16, 16)]
            idx_v[pl.ds(j * 16, 16)] = p & _SRC_MASK
            loc_v[pl.ds(j * 16, 16)] = lax.shift_right_logical(p, 14)
        pltpu.async_copy(h_hbm.at[idx_v], rows_v, sem).wait()

        def edge16(g, c2):
            loc16 = loc_v[pl.ds(g * 16, 16)] * F
            for k in range(16):
                base = loc16[k]
                for f in range(F // 16):
                    a = acc_v[pl.ds(base + f * 16, 16)]
                    r = rows_v[g * 16 + k, pl.ds(f * 16, 16)]
                    acc_v[pl.ds(base + f * 16, 16)] = jnp.maximum(a, r)
            return c2

        lax.fori_loop(0, G // 16, edge16, 0)
        return carry

    lax.fori_loop(0, nb, batch, 0)
    pltpu.sync_copy(acc_v.at[pl.ds(0, NP * F)],
                    out_hbm.at[pl.ds(pl.multiple_of(w * NP * F, 8), NP * F)])


def _make_agg(F):
    mesh = plsc.VectorSubcoreMesh(core_axis_name="c", subcore_axis_name="s")
    return pl.kernel(
        functools.partial(_agg_body, F),
        out_type=jax.ShapeDtypeStruct((N_PAD * F,), jnp.float32),
        mesh=mesh,
        compiler_params=pltpu.CompilerParams(needs_layout_passes=False,
                                             use_tc_tiling_on_sc=False),
        scratch_types=[
            pltpu.VMEM((G,), jnp.int32),
            pltpu.VMEM((G,), jnp.int32),
            pltpu.VMEM((G,), jnp.int32),
            pltpu.VMEM((G, F), jnp.float32),
            pltpu.VMEM(((NP + 1) * F,), jnp.float32),
            pltpu.VMEM((16,), jnp.int32),
            pltpu.SemaphoreType.DMA,
        ],
    )


def _mm_body(relu, logsoftmax, agg_ref, h_ref, wl_ref, bl_ref, wr_ref, o_ref):
    agg = agg_ref[...]
    aggf = jnp.where(agg == -jnp.inf, 0.0, agg)
    z = lax.dot_general(aggf, wl_ref[...], (((1,), (1,)), ((), ())),
                        preferred_element_type=jnp.float32)
    z = z + bl_ref[...]
    z = z + lax.dot_general(h_ref[...], wr_ref[...], (((1,), (1,)), ((), ())),
                            preferred_element_type=jnp.float32)
    if relu:
        z = jnp.maximum(z, 0.0)
    if logsoftmax:
        zmax = jnp.max(z, axis=1, keepdims=True)
        zs = z - zmax
        z = zs - jnp.log(jnp.sum(jnp.exp(zs), axis=1, keepdims=True))
    o_ref[...] = z


def _mm(agg, h, Wl, bl, Wr, relu, logsoftmax):
    Fo = Wl.shape[0]
    return pl.pallas_call(
        functools.partial(_mm_body, relu, logsoftmax),
        out_shape=jax.ShapeDtypeStruct((N_PAD, Fo), jnp.float32),
    )(agg, h, Wl, bl.reshape(1, Fo), Wr)


def kernel(x, edge_index, Wl1, bl1, Wr1, Wl2, bl2, Wr2, Wl3, bl3, Wr3):
    src = edge_index[0]
    dst = edge_index[1]
    x_pad = jnp.zeros((N_PAD, D), jnp.float32).at[:N].set(x)

    bucket, counts = _bucket(dst, src)

    agg1 = _make_agg(D)(x_pad, bucket, counts).reshape(N_PAD, D)
    h1 = _mm(agg1, x_pad, Wl1, bl1, Wr1, relu=True, logsoftmax=False)

    agg2 = _make_agg(H)(h1, bucket, counts).reshape(N_PAD, H)
    h2 = _mm(agg2, h1, Wl2, bl2, Wr2, relu=True, logsoftmax=False)

    agg3 = _make_agg(H)(h2, bucket, counts).reshape(N_PAD, H)
    h3 = _mm(agg3, h2, Wl3, bl3, Wr3, relu=False, logsoftmax=True)

    return h3[:N]


# speedup vs baseline: 3.9873x; 1.1546x over previous
import functools

import jax
import jax.numpy as jnp
from jax import lax
from jax.experimental import pallas as pl
from jax.experimental.pallas import tpu as pltpu
from jax.experimental.pallas import tpu_sc as plsc

N = 10000
E = 320000
D = 128
H = 64

NC = 2
NS = 16
NW = NC * NS
NP = 313
N_PAD = NW * NP

CH = 10000
G = 256
CAP = E + 2 * CH

_SRC_MASK = (1 << 14) - 1


def _worker_id():
    return lax.axis_index("s") * NC + lax.axis_index("c")


def _memset_i32(ref, n, value):
    v = jnp.full((16,), value, jnp.int32)

    def body(i, carry):
        ref[pl.ds(i * 16, 16)] = v
        return carry

    lax.fori_loop(0, n // 16, body, 0)


def _bucket_body(dst_hbm, src_hbm, bucket_hbm, counts_hbm,
                 dst_v0, dst_v1, src_v0, src_v1, sel_v, cnt_v, sems):
    dst_b = (dst_v0, dst_v1)
    src_b = (src_v0, src_v1)
    w = _worker_id()
    lo = w * NP
    hi = lo + NP
    lo_v = jnp.full((16,), lo, jnp.int32)
    hi_v = jnp.full((16,), hi, jnp.int32)
    trash = (NP << 14) | lo
    _memset_i32(sel_v, CH + 16, trash)

    def load(c, u):
        base = pl.ds(pl.multiple_of(c * CH, 8), CH)
        pltpu.async_copy(dst_hbm.at[base], dst_b[u], sems.at[2 * u])
        pltpu.async_copy(src_hbm.at[base], src_b[u], sems.at[2 * u + 1])

    def wait(c, u):
        base = pl.ds(pl.multiple_of(c * CH, 8), CH)
        pltpu.make_async_copy(dst_hbm.at[base], dst_b[u], sems.at[2 * u]).wait()
        pltpu.make_async_copy(src_hbm.at[base], src_b[u], sems.at[2 * u + 1]).wait()

    def filter_chunk(c, u, off):
        def filt(i, cnt):
            d = dst_b[u][pl.ds(i * 16, 16)]
            s = src_b[u][pl.ds(i * 16, 16)]
            m = (d >= lo_v) & (d < hi_v)
            packed = ((d - lo_v) << 14) | s
            pos = plsc.cumsum(m.astype(jnp.int32)) + (cnt - 1)
            plsc.store_scatter(sel_v, [pos], packed, mask=m)
            pc = plsc.all_reduce_population_count(m)
            return cnt + pc[0]

        cnt = lax.fori_loop(0, CH // 16, filt, 0)
        sel_v[pl.ds(cnt, 16)] = jnp.full((16,), trash, jnp.int32)
        cnt_p = (cnt + 7) & ~7
        pltpu.sync_copy(sel_v.at[pl.ds(0, CH)],
                        bucket_hbm.at[pl.ds(pl.multiple_of(w * CAP + off, 8), CH)])
        return off + cnt_p

    load(0, 0)

    def pair(i, off):
        c0 = i * 2
        load(c0 + 1, 1)
        wait(c0, 0)
        off = filter_chunk(c0, 0, off)
        load(c0 + 2, 0)
        wait(c0 + 1, 1)
        off = filter_chunk(c0 + 1, 1, off)
        return off

    off = lax.fori_loop(0, E // CH // 2 - 1, pair, 0)
    c0 = E // CH - 2
    load(c0 + 1, 1)
    wait(c0, 0)
    off = filter_chunk(c0, 0, off)
    wait(c0 + 1, 1)
    off = filter_chunk(c0 + 1, 1, off)
    pltpu.sync_copy(sel_v.at[pl.ds(0, CH)], bucket_hbm.at[pl.ds(pl.multiple_of(w * CAP + off, 8), CH)])
    cnt_v[...] = jnp.full((16,), off, jnp.int32)
    pltpu.sync_copy(cnt_v, counts_hbm.at[pl.ds(pl.multiple_of(w * 16, 8), 16)])


@functools.partial(jax.jit, static_argnums=())
def _bucket(dst, src):
    mesh = plsc.VectorSubcoreMesh(core_axis_name="c", subcore_axis_name="s")
    kern = pl.kernel(
        _bucket_body,
        out_type=[
            jax.ShapeDtypeStruct((NW * CAP,), jnp.int32),
            jax.ShapeDtypeStruct((NW * 16,), jnp.int32),
        ],
        mesh=mesh,
        compiler_params=pltpu.CompilerParams(needs_layout_passes=False),
        scratch_types=[
            pltpu.VMEM((CH,), jnp.int32),
            pltpu.VMEM((CH,), jnp.int32),
            pltpu.VMEM((CH,), jnp.int32),
            pltpu.VMEM((CH,), jnp.int32),
            pltpu.VMEM((CH + 16,), jnp.int32),
            pltpu.VMEM((16,), jnp.int32),
            pltpu.SemaphoreType.DMA((4,)),
        ],
    )
    return kern(dst, src)


def _agg_body(F, h_hbm, bucket_hbm, counts_hbm, out_hbm,
              pk_v, idx_v0, idx_v1, loc_v0, loc_v1, rows_v0, rows_v1,
              acc_v, cnt_v, sems):
    idx_b = (idx_v0, idx_v1)
    loc_b = (loc_v0, loc_v1)
    rows_b = (rows_v0, rows_v1)
    w = _worker_id()
    pltpu.sync_copy(counts_hbm.at[pl.ds(pl.multiple_of(w * 16, 8), 16)], cnt_v)
    cnt = jnp.max(cnt_v[...])

    neginf = jnp.full((16,), -jnp.inf, jnp.float32)

    def init(i, carry):
        acc_v[pl.ds(i * 16, 16)] = neginf
        return carry

    lax.fori_loop(0, (NP + 1) * F // 16, init, 0)

    def prep(b, u):
        pltpu.sync_copy(
            bucket_hbm.at[pl.ds(pl.multiple_of(w * CAP + b * G, 8), G)], pk_v)
        for j in range(G // 16):
            p = pk_v[pl.ds(j * 16, 16)]
            idx_b[u][pl.ds(j * 16, 16)] = p & _SRC_MASK
            loc_b[u][pl.ds(j * 16, 16)] = lax.shift_right_logical(p, 14)
        pltpu.async_copy(h_hbm.at[idx_b[u]], rows_b[u], sems.at[u])

    def wait(u):
        pltpu.make_async_copy(h_hbm.at[idx_b[u]], rows_b[u],
                              sems.at[u]).wait()

    def process(u):
        def edge16(g, c2):
            loc16 = loc_b[u][pl.ds(g * 16, 16)] * F
            for k in range(16):
                base = loc16[k]
                for f in range(F // 16):
                    a = acc_v[pl.ds(base + f * 16, 16)]
                    r = rows_b[u][g * 16 + k, pl.ds(f * 16, 16)]
                    acc_v[pl.ds(base + f * 16, 16)] = jnp.maximum(a, r)
            return c2

        lax.fori_loop(0, G // 16, edge16, 0)

    prep(0, 0)
    nb2 = lax.div(cnt + (2 * G - 1), 2 * G)

    def pairb(i, carry):
        b0 = i * 2
        prep(b0 + 1, 1)
        wait(0)
        process(0)
        prep(b0 + 2, 0)
        wait(1)
        process(1)
        return carry

    lax.fori_loop(0, nb2, pairb, 0)
    wait(0)
    pltpu.sync_copy(acc_v.at[pl.ds(0, NP * F)],
                    out_hbm.at[pl.ds(pl.multiple_of(w * NP * F, 8), NP * F)])


def _make_agg(F):
    mesh = plsc.VectorSubcoreMesh(core_axis_name="c", subcore_axis_name="s")
    return pl.kernel(
        functools.partial(_agg_body, F),
        out_type=jax.ShapeDtypeStruct((N_PAD * F,), jnp.float32),
        mesh=mesh,
        compiler_params=pltpu.CompilerParams(needs_layout_passes=False,
                                             use_tc_tiling_on_sc=False),
        scratch_types=[
            pltpu.VMEM((G,), jnp.int32),
            pltpu.VMEM((G,), jnp.int32),
            pltpu.VMEM((G,), jnp.int32),
            pltpu.VMEM((G,), jnp.int32),
            pltpu.VMEM((G,), jnp.int32),
            pltpu.VMEM((G, F), jnp.float32),
            pltpu.VMEM((G, F), jnp.float32),
            pltpu.VMEM(((NP + 1) * F,), jnp.float32),
            pltpu.VMEM((16,), jnp.int32),
            pltpu.SemaphoreType.DMA((2,)),
        ],
    )


def _mm_body(relu, logsoftmax, agg_ref, h_ref, wl_ref, bl_ref, wr_ref, o_ref):
    agg = agg_ref[...]
    aggf = jnp.where(agg == -jnp.inf, 0.0, agg)
    z = lax.dot_general(aggf, wl_ref[...], (((1,), (1,)), ((), ())),
                        preferred_element_type=jnp.float32)
    z = z + bl_ref[...]
    z = z + lax.dot_general(h_ref[...], wr_ref[...], (((1,), (1,)), ((), ())),
                            preferred_element_type=jnp.float32)
    if relu:
        z = jnp.maximum(z, 0.0)
    if logsoftmax:
        zmax = jnp.max(z, axis=1, keepdims=True)
        zs = z - zmax
        z = zs - jnp.log(jnp.sum(jnp.exp(zs), axis=1, keepdims=True))
    o_ref[...] = z


def _mm(agg, h, Wl, bl, Wr, relu, logsoftmax):
    Fo = Wl.shape[0]
    return pl.pallas_call(
        functools.partial(_mm_body, relu, logsoftmax),
        out_shape=jax.ShapeDtypeStruct((N_PAD, Fo), jnp.float32),
    )(agg, h, Wl, bl.reshape(1, Fo), Wr)


def kernel(x, edge_index, Wl1, bl1, Wr1, Wl2, bl2, Wr2, Wl3, bl3, Wr3):
    src = edge_index[0]
    dst = edge_index[1]
    x_pad = jnp.zeros((N_PAD, D), jnp.float32).at[:N].set(x)

    bucket, counts = _bucket(dst, src)

    agg1 = _make_agg(D)(x_pad, bucket, counts).reshape(N_PAD, D)
    h1 = _mm(agg1, x_pad, Wl1, bl1, Wr1, relu=True, logsoftmax=False)

    agg2 = _make_agg(H)(h1, bucket, counts).reshape(N_PAD, H)
    h2 = _mm(agg2, h1, Wl2, bl2, Wr2, relu=True, logsoftmax=False)

    agg3 = _make_agg(H)(h2, bucket, counts).reshape(N_PAD, H)
    h3 = _mm(agg3, h2, Wl3, bl3, Wr3, relu=False, logsoftmax=True)

    return h3[:N]
